# Initial kernel scaffold; baseline (speedup 1.0000x reference)
import jax
import jax.numpy as jnp
from jax.experimental import pallas as pl


def kernel(node_features, edge_features, edge_idx, batch_idx, params):
    raise NotImplementedError("write your pallas kernel here")


# speedup vs baseline: 4.7725x; 4.7725x over previous
import functools

import jax
import jax.numpy as jnp
from jax import lax
from jax.experimental import pallas as pl
from jax.experimental.pallas import tpu as pltpu
from jax.experimental.pallas import tpu_sc as plsc

_NC = 2
_NS = 16
_NW = _NC * _NS
_CH = 1024
_ST = 128
_K = _CH // _ST

@functools.cache
def _mesh():
    return plsc.VectorSubcoreMesh(
        core_axis_name="c", subcore_axis_name="s",
        num_cores=_NC, num_subcores=_NS,
    )


def _wid():
    return lax.axis_index("s") * _NC + lax.axis_index("c")


@functools.cache
def _gather_fn(n_edges, dim, n_rows):
    nblk = n_edges // _CH
    base_blocks = nblk // _NW
    extra = nblk % _NW

    @functools.partial(
        pl.kernel,
        out_type=jax.ShapeDtypeStruct((n_edges, dim), jnp.float32),
        mesh=_mesh(),
        compiler_params=pltpu.CompilerParams(use_tc_tiling_on_sc=False),
        scratch_types=[
            pltpu.VMEM((_CH,), jnp.int32),
            pltpu.VMEM((_CH, dim), jnp.float32),
            pltpu.SemaphoreType.DMA,
        ],
    )
    def gather_k(table, idx, out, idx_v, rows_v, sem):
        w = _wid()
        nb = base_blocks + jnp.where(w < extra, 1, 0)

        def body(i, carry):
            blk = w + i * _NW
            off = blk * _CH
            pltpu.sync_copy(idx.at[pl.ds(off, _CH)], idx_v)
            cps = [
                pltpu.async_copy(
                    table.at[idx_v.at[pl.ds(j * _ST, _ST)]],
                    rows_v.at[pl.ds(j * _ST, _ST)],
                    sem,
                )
                for j in range(_K)
            ]
            for cp in cps:
                cp.wait()
            pltpu.sync_copy(rows_v, out.at[pl.ds(off, _CH)])
            return carry

        lax.fori_loop(0, nb, body, 0)

    return gather_k


@functools.cache
def _scatter_fn(n_edges, dim, n_rows):
    nblk = n_edges // _CH
    base_blocks = nblk // _NW
    extra = nblk % _NW
    rows_per_tile = n_rows // _NS
    zch = min(rows_per_tile, 250)
    n_zch = rows_per_tile // zch

    @functools.partial(
        pl.kernel,
        out_type=jax.ShapeDtypeStruct((_NC, n_rows, dim), jnp.float32),
        mesh=_mesh(),
        compiler_params=pltpu.CompilerParams(use_tc_tiling_on_sc=False),
        scratch_types=[
            pltpu.VMEM((_K, _ST), jnp.int32),
            pltpu.VMEM((_CH, dim), jnp.float32),
            pltpu.VMEM((zch, dim), jnp.float32),
            pltpu.VMEM_SHARED((n_rows, dim), jnp.float32),
            pltpu.SemaphoreType.DMA,
        ],
    )
    def scatter_k(rows, idx2, out, idx_v, rows_v, zbuf, table, sem):
        c = lax.axis_index("c")
        s = lax.axis_index("s")
        w = _wid()

        def zero_body(i, carry):
            zbuf[i, :] = jnp.zeros((16,), jnp.float32)
            return carry

        lax.fori_loop(0, zch, zero_body, 0)
        for q in range(n_zch):
            pltpu.sync_copy(
                zbuf, table.at[pl.ds(s * rows_per_tile + q * zch, zch)]
            )
        plsc.subcore_barrier()

        nb = base_blocks + jnp.where(w < extra, 1, 0)

        def body(i, carry):
            blk = w + i * _NW
            pltpu.sync_copy(idx2.at[pl.ds(blk * _K, _K)], idx_v)
            pltpu.sync_copy(rows.at[pl.ds(blk * _CH, _CH)], rows_v)
            for j in range(_K):
                pltpu.sync_copy(
                    rows_v.at[pl.ds(j * _ST, _ST)],
                    table.at[idx_v.at[j]],
                    add=True,
                )
            return carry

        lax.fori_loop(0, nb, body, 0)
        plsc.subcore_barrier()
        for q in range(n_zch):
            r0 = s * rows_per_tile + q * zch
            pltpu.sync_copy(table.at[pl.ds(r0, zch)], zbuf)
            pltpu.sync_copy(zbuf, out.at[c].at[pl.ds(r0, zch)])

    return scatter_k


def _dotd(a, b):
    return jnp.dot(
        a.astype(jnp.bfloat16), b.astype(jnp.bfloat16),
        preferred_element_type=jnp.float32,
    )


def _proj_body(x_ref, w_ref, b_ref, out_ref):
    out_ref[...] = _dotd(x_ref[...], w_ref[...]) + b_ref[...]


def _proj(x, w, b):
    n, din = x.shape
    dout = w.shape[1]
    bn = 4000
    return pl.pallas_call(
        _proj_body,
        grid=(n // bn,),
        in_specs=[
            pl.BlockSpec((bn, din), lambda i: (i, 0)),
            pl.BlockSpec((din, dout), lambda i: (0, 0)),
            pl.BlockSpec((1, dout), lambda i: (0, 0)),
        ],
        out_specs=pl.BlockSpec((bn, dout), lambda i: (i, 0)),
        out_shape=jax.ShapeDtypeStruct((n, dout), jnp.float32),
    )(x, w, b)


def _msg_body(g_ref, ef_ref, w1e_ref, w2_ref, b2_ref, out_ref):
    h1 = jnp.maximum(g_ref[...] + _dotd(ef_ref[...], w1e_ref[...]), 0.0)
    out_ref[...] = jnp.maximum(_dotd(h1, w2_ref[...]) + b2_ref[...], 0.0)


def _msg(g, ef, w1e, w2, b2):
    e, dm = g.shape
    de = ef.shape[1]
    be = 6400
    return pl.pallas_call(
        _msg_body,
        grid=(e // be,),
        in_specs=[
            pl.BlockSpec((be, dm), lambda i: (i, 0)),
            pl.BlockSpec((be, de), lambda i: (i, 0)),
            pl.BlockSpec((de, dm), lambda i: (0, 0)),
            pl.BlockSpec((dm, dm), lambda i: (0, 0)),
            pl.BlockSpec((1, dm), lambda i: (0, 0)),
        ],
        out_specs=pl.BlockSpec((be, dm), lambda i: (i, 0)),
        out_shape=jax.ShapeDtypeStruct((e, dm), jnp.float32),
    )(g, ef, w1e, w2, b2)


def _upd_body(x_ref, a_ref, ux_ref, ua_ref, b_ref, out_ref):
    a = a_ref[0] + a_ref[1]
    out_ref[...] = jnp.maximum(
        _dotd(x_ref[...], ux_ref[...]) + _dotd(a, ua_ref[...]) + b_ref[...],
        0.0,
    )


def _update(x, aggp, ux, ua, b):
    n, din = x.shape
    dm = ua.shape[0]
    dout = ux.shape[1]
    bn = 4000
    return pl.pallas_call(
        _upd_body,
        grid=(n // bn,),
        in_specs=[
            pl.BlockSpec((bn, din), lambda i: (i, 0)),
            pl.BlockSpec((2, bn, dm), lambda i: (0, i, 0)),
            pl.BlockSpec((din, dout), lambda i: (0, 0)),
            pl.BlockSpec((dm, dout), lambda i: (0, 0)),
            pl.BlockSpec((1, dout), lambda i: (0, 0)),
        ],
        out_specs=pl.BlockSpec((bn, dout), lambda i: (i, 0)),
        out_shape=jax.ShapeDtypeStruct((n, dout), jnp.float32),
    )(x, aggp, ux, ua, b)


def _pool_readout_body(
    y_ref, b_ref, wh_ref, bh_ref, wo_ref, bo_ref, out_ref, acc_ref
):
    i = pl.program_id(0)
    ng = acc_ref.shape[0]

    @pl.when(i == 0)
    def _():
        acc_ref[...] = jnp.zeros_like(acc_ref)

    bids = b_ref[0, 0, :]
    onehot_t = jnp.where(
        lax.broadcasted_iota(jnp.int32, (ng, bids.shape[0]), 0)
        == bids[None, :],
        1.0,
        0.0,
    )
    acc_ref[...] += jnp.dot(
        onehot_t, y_ref[...],
        preferred_element_type=jnp.float32, precision=lax.Precision.HIGHEST,
    )

    @pl.when(i == pl.num_programs(0) - 1)
    def _():
        h = jnp.maximum(_dotd(acc_ref[...], wh_ref[...]) + bh_ref[...], 0.0)
        out_ref[...] = _dotd(h, wo_ref[...]) + bo_ref[...]


def _pool_readout(y, batch3, wh, bh, wo, bo, n_graphs):
    n, dm = y.shape
    bn = 4000
    dh = wh.shape[1]
    return pl.pallas_call(
        _pool_readout_body,
        grid=(n // bn,),
        in_specs=[
            pl.BlockSpec((bn, dm), lambda i: (i, 0)),
            pl.BlockSpec((1, 1, bn), lambda i: (i, 0, 0)),
            pl.BlockSpec((dm, dh), lambda i: (0, 0)),
            pl.BlockSpec((1, dh), lambda i: (0, 0)),
            pl.BlockSpec((dh, 1), lambda i: (0, 0)),
            pl.BlockSpec((1, 1), lambda i: (0, 0)),
        ],
        out_specs=pl.BlockSpec((n_graphs, 1), lambda i: (0, 0)),
        out_shape=jax.ShapeDtypeStruct((n_graphs, 1), jnp.float32),
        scratch_shapes=[pltpu.VMEM((n_graphs, dm), jnp.float32)],
    )(y, batch3, wh, bh, wo, bo)


def kernel(node_features, edge_features, edge_idx, batch_idx, params):
    n_nodes, _ = node_features.shape
    n_edges = edge_features.shape[0]
    n_graphs = 128
    src = edge_idx[0]
    dst2 = edge_idx[1].reshape(n_edges // _ST, _ST)

    y = node_features
    for p in params["gnn"]:
        (w1, b1), (w2, b2) = p["M"]
        uw, ub = p["U"][0]
        din = y.shape[1]
        w1x, w1e = w1[:din], w1[din:]
        proj = _proj(y, w1x, b1.reshape(1, -1))
        g = _gather_fn(n_edges, proj.shape[1], n_nodes)(proj, src)
        h = _msg(g, edge_features, w1e, w2, b2.reshape(1, -1))
        aggp = _scatter_fn(n_edges, h.shape[1], n_nodes)(h, dst2)
        y = _update(y, aggp, uw[:din], uw[din:], ub.reshape(1, -1))

    batch3 = batch_idx.astype(jnp.int32).reshape(n_nodes // 4000, 1, 4000)
    return _pool_readout(
        y,
        batch3,
        params["mlp_h_w"],
        params["mlp_h_b"].reshape(1, -1),
        params["mlp_o_w"],
        params["mlp_o_b"].reshape(1, -1),
        n_graphs,
    )


# speedup vs baseline: 5.0245x; 1.0528x over previous
import functools

import jax
import jax.numpy as jnp
from jax import lax
from jax.experimental import pallas as pl
from jax.experimental.pallas import tpu as pltpu
from jax.experimental.pallas import tpu_sc as plsc

_NC = 2
_NS = 16
_NW = _NC * _NS
_CH = 1024
_ST = 128
_K = _CH // _ST

@functools.cache
def _mesh():
    return plsc.VectorSubcoreMesh(
        core_axis_name="c", subcore_axis_name="s",
        num_cores=_NC, num_subcores=_NS,
    )


def _wid():
    return lax.axis_index("s") * _NC + lax.axis_index("c")


@functools.cache
def _gather_fn(n_edges, dim, n_rows):
    nblk = n_edges // _CH
    base_blocks = nblk // _NW
    extra = nblk % _NW

    @functools.partial(
        pl.kernel,
        out_type=jax.ShapeDtypeStruct((n_edges, dim), jnp.float32),
        mesh=_mesh(),
        compiler_params=pltpu.CompilerParams(use_tc_tiling_on_sc=False),
        scratch_types=[
            pltpu.VMEM((_CH,), jnp.int32),
            pltpu.VMEM((_CH, dim), jnp.float32),
            pltpu.SemaphoreType.DMA,
        ],
    )
    def gather_k(table, idx, out, idx_v, rows_v, sem):
        w = _wid()
        nb = base_blocks + jnp.where(w < extra, 1, 0)

        def body(i, carry):
            blk = w + i * _NW
            off = blk * _CH
            pltpu.sync_copy(idx.at[pl.ds(off, _CH)], idx_v)
            cps = [
                pltpu.async_copy(
                    table.at[idx_v.at[pl.ds(j * _ST, _ST)]],
                    rows_v.at[pl.ds(j * _ST, _ST)],
                    sem,
                )
                for j in range(_K)
            ]
            for cp in cps:
                cp.wait()
            pltpu.sync_copy(rows_v, out.at[pl.ds(off, _CH)])
            return carry

        lax.fori_loop(0, nb, body, 0)

    return gather_k


@functools.cache
def _scatter_fn(n_edges, dim, n_rows):
    nblk = n_edges // _CH
    base_blocks = nblk // _NW
    extra = nblk % _NW
    rows_per_tile = n_rows // _NS
    zch = 320
    while rows_per_tile % zch:
        zch //= 2
    zch = min(rows_per_tile, zch)
    n_zch = rows_per_tile // zch

    @functools.partial(
        pl.kernel,
        out_type=jax.ShapeDtypeStruct((_NC, n_rows, dim), jnp.float32),
        mesh=_mesh(),
        compiler_params=pltpu.CompilerParams(use_tc_tiling_on_sc=False),
        scratch_types=[
            pltpu.VMEM((_K, _ST), jnp.int32),
            pltpu.VMEM((_CH, dim), jnp.float32),
            pltpu.VMEM((zch, dim), jnp.float32),
            pltpu.VMEM_SHARED((n_rows, dim), jnp.float32),
            pltpu.SemaphoreType.DMA,
        ],
    )
    def scatter_k(rows, idx2, out, idx_v, rows_v, zbuf, table, sem):
        c = lax.axis_index("c")
        s = lax.axis_index("s")
        w = _wid()

        def zero_body(i, carry):
            zbuf[i, :] = jnp.zeros((16,), jnp.float32)
            return carry

        lax.fori_loop(0, zch, zero_body, 0)
        for q in range(n_zch):
            pltpu.sync_copy(
                zbuf, table.at[pl.ds(s * rows_per_tile + q * zch, zch)]
            )
        plsc.subcore_barrier()

        nb = base_blocks + jnp.where(w < extra, 1, 0)

        def body(i, carry):
            blk = w + i * _NW
            pltpu.sync_copy(idx2.at[pl.ds(blk * _K, _K)], idx_v)
            pltpu.sync_copy(rows.at[pl.ds(blk * _CH, _CH)], rows_v)
            for j in range(_K):
                pltpu.sync_copy(
                    rows_v.at[pl.ds(j * _ST, _ST)],
                    table.at[idx_v.at[j]],
                    add=True,
                )
            return carry

        lax.fori_loop(0, nb, body, 0)
        plsc.subcore_barrier()
        for q in range(n_zch):
            r0 = s * rows_per_tile + q * zch
            pltpu.sync_copy(table.at[pl.ds(r0, zch)], zbuf)
            pltpu.sync_copy(zbuf, out.at[c].at[pl.ds(r0, zch)])

    return scatter_k


def _dotd(a, b):
    return jnp.dot(
        a.astype(jnp.bfloat16), b.astype(jnp.bfloat16),
        preferred_element_type=jnp.float32,
    )


def _proj_body(x_ref, w_ref, b_ref, out_ref):
    out_ref[...] = _dotd(x_ref[...], w_ref[...]) + b_ref[...]


def _proj(x2, wbd, bt):
    n8, dk = x2.shape
    bn = 3200
    return pl.pallas_call(
        _proj_body,
        grid=(n8 // bn,),
        in_specs=[
            pl.BlockSpec((bn, dk), lambda i: (i, 0)),
            pl.BlockSpec((dk, 128), lambda i: (0, 0)),
            pl.BlockSpec((1, 128), lambda i: (0, 0)),
        ],
        out_specs=pl.BlockSpec((bn, 128), lambda i: (i, 0)),
        out_shape=jax.ShapeDtypeStruct((n8, 128), jnp.float32),
    )(x2, wbd, bt)


def _msg_body(g_ref, ef_ref, w1e_ref, w2_ref, b2_ref, out_ref):
    h1 = jnp.maximum(g_ref[...] + _dotd(ef_ref[...], w1e_ref[...]), 0.0)
    out_ref[...] = jnp.maximum(_dotd(h1, w2_ref[...]) + b2_ref[...], 0.0)


def _msg(g2, ef2, w1ebd, w2bd, b2t):
    e8, _ = g2.shape
    de = ef2.shape[1]
    be = 2000
    return pl.pallas_call(
        _msg_body,
        grid=(e8 // be,),
        in_specs=[
            pl.BlockSpec((be, 128), lambda i: (i, 0)),
            pl.BlockSpec((be, de), lambda i: (i, 0)),
            pl.BlockSpec((de, 128), lambda i: (0, 0)),
            pl.BlockSpec((128, 128), lambda i: (0, 0)),
            pl.BlockSpec((1, 128), lambda i: (0, 0)),
        ],
        out_specs=pl.BlockSpec((be, 128), lambda i: (i, 0)),
        out_shape=jax.ShapeDtypeStruct((e8, 128), jnp.float32),
    )(g2, ef2, w1ebd, w2bd, b2t)


def _upd_body(x_ref, a_ref, ux_ref, ua_ref, b_ref, out_ref):
    a = a_ref[0] + a_ref[1]
    out_ref[...] = jnp.maximum(
        _dotd(x_ref[...], ux_ref[...]) + _dotd(a, ua_ref[...]) + b_ref[...],
        0.0,
    )


def _update(x2, aggp2, uxbd, uabd, bt):
    n8, dk = x2.shape
    bn = 3200
    return pl.pallas_call(
        _upd_body,
        grid=(n8 // bn,),
        in_specs=[
            pl.BlockSpec((bn, dk), lambda i: (i, 0)),
            pl.BlockSpec((2, bn, 128), lambda i: (0, i, 0)),
            pl.BlockSpec((dk, 128), lambda i: (0, 0)),
            pl.BlockSpec((128, 128), lambda i: (0, 0)),
            pl.BlockSpec((1, 128), lambda i: (0, 0)),
        ],
        out_specs=pl.BlockSpec((bn, 128), lambda i: (i, 0)),
        out_shape=jax.ShapeDtypeStruct((n8, 128), jnp.float32),
    )(x2, aggp2, uxbd, uabd, bt)


def _pool_readout_body(
    y_ref, b_ref, wh_ref, bh_ref, wo_ref, bo_ref, out_ref, acc_ref
):
    i = pl.program_id(0)
    ng = acc_ref.shape[0]

    @pl.when(i == 0)
    def _():
        acc_ref[...] = jnp.zeros_like(acc_ref)

    bids = b_ref[0, 0, :]
    onehot_t = jnp.where(
        lax.broadcasted_iota(jnp.int32, (ng, bids.shape[0]), 0)
        == bids[None, :],
        1.0,
        0.0,
    )
    acc_ref[...] += jnp.dot(
        onehot_t, y_ref[...],
        preferred_element_type=jnp.float32, precision=lax.Precision.HIGHEST,
    )

    @pl.when(i == pl.num_programs(0) - 1)
    def _():
        h = jnp.maximum(_dotd(acc_ref[...], wh_ref[...]) + bh_ref[...], 0.0)
        out_ref[...] = _dotd(h, wo_ref[...]) + bo_ref[...]


def _pool_readout(y, batch3, wh, bh, wo, bo, n_graphs):
    n, dm = y.shape
    bn = 4096
    dh = wh.shape[1]
    return pl.pallas_call(
        _pool_readout_body,
        grid=(n // bn,),
        in_specs=[
            pl.BlockSpec((bn, dm), lambda i: (i, 0)),
            pl.BlockSpec((1, 1, bn), lambda i: (i, 0, 0)),
            pl.BlockSpec((dm, dh), lambda i: (0, 0)),
            pl.BlockSpec((1, dh), lambda i: (0, 0)),
            pl.BlockSpec((dh, 1), lambda i: (0, 0)),
            pl.BlockSpec((1, 1), lambda i: (0, 0)),
        ],
        out_specs=pl.BlockSpec((n_graphs, 1), lambda i: (0, 0)),
        out_shape=jax.ShapeDtypeStruct((n_graphs, 1), jnp.float32),
        scratch_shapes=[pltpu.VMEM((n_graphs, dm), jnp.float32)],
    )(y, batch3, wh, bh, wo, bo)


def kernel(node_features, edge_features, edge_idx, batch_idx, params):
    n_nodes, _ = node_features.shape
    n_edges = edge_features.shape[0]
    n_graphs = 128
    n_pad = 102400
    src = edge_idx[0]
    dst2 = edge_idx[1].reshape(n_edges // _ST, _ST)
    ef2 = edge_features.reshape(n_edges // 8, 24)

    def bd(w):
        k = w.shape[0]
        out = jnp.zeros((8, k, 8, 16), jnp.float32)
        out = out.at[jnp.arange(8), :, jnp.arange(8), :].set(
            jnp.broadcast_to(w, (8, k, 16)))
        return out.reshape(8 * k, 128)

    def tile_b(b):
        return jnp.tile(b.reshape(1, 16), (1, 8))

    x_pad = jnp.concatenate(
        [node_features,
         jnp.zeros((n_pad - n_nodes, node_features.shape[1]), jnp.float32)])
    y2 = x_pad.reshape(n_pad // 8, -1)
    for p in params["gnn"]:
        (w1, b1), (w2, b2) = p["M"]
        uw, ub = p["U"][0]
        din = y2.shape[1] // 8
        w1x, w1e = w1[:din], w1[din:]
        proj2 = _proj(y2, bd(w1x), tile_b(b1))
        g = _gather_fn(n_edges, 16, n_pad)(proj2.reshape(n_pad, 16), src)
        h2 = _msg(g.reshape(n_edges // 8, 128), ef2,
                  bd(w1e), bd(w2), tile_b(b2))
        aggp = _scatter_fn(n_edges, 16, n_pad)(h2.reshape(n_edges, 16), dst2)
        y2 = _update(y2, aggp.reshape(2, n_pad // 8, 128),
                     bd(uw[:din]), bd(uw[din:]), tile_b(ub))

    bpad = jnp.concatenate(
        [batch_idx.astype(jnp.int32),
         jnp.full((n_pad - n_nodes,), n_graphs, jnp.int32)])
    batch3 = bpad.reshape(n_pad // 4096, 1, 4096)
    return _pool_readout(
        y2.reshape(n_pad, 16),
        batch3,
        params["mlp_h_w"],
        params["mlp_h_b"].reshape(1, -1),
        params["mlp_o_w"],
        params["mlp_o_b"].reshape(1, -1),
        n_graphs,
    )


# speedup vs baseline: 5.2028x; 1.0355x over previous
import functools

import jax
import jax.numpy as jnp
from jax import lax
from jax.experimental import pallas as pl
from jax.experimental.pallas import tpu as pltpu
from jax.experimental.pallas import tpu_sc as plsc

_NC = 2
_NS = 16
_NW = _NC * _NS
_CH = 1024
_ST = 128
_K = _CH // _ST

@functools.cache
def _mesh():
    return plsc.VectorSubcoreMesh(
        core_axis_name="c", subcore_axis_name="s",
        num_cores=_NC, num_subcores=_NS,
    )


def _wid():
    return lax.axis_index("s") * _NC + lax.axis_index("c")


@functools.cache
def _gather_fn(n_edges, dim, n_rows):
    nblk = n_edges // _CH
    base_blocks = nblk // _NW
    extra = nblk % _NW

    @functools.partial(
        pl.kernel,
        out_type=jax.ShapeDtypeStruct((n_edges, dim), jnp.float32),
        mesh=_mesh(),
        compiler_params=pltpu.CompilerParams(use_tc_tiling_on_sc=False),
        scratch_types=[
            pltpu.VMEM((2, _CH), jnp.int32),
            pltpu.VMEM((2, _CH, dim), jnp.float32),
            pltpu.SemaphoreType.DMA,
            pltpu.SemaphoreType.DMA,
            pltpu.SemaphoreType.DMA,
            pltpu.SemaphoreType.DMA,
            pltpu.SemaphoreType.DMA,
        ],
    )
    def gather_k(table, idx, out, idx_v, rows_v, sia, sib, sg, swa, swb):
        w = _wid()
        nb = base_blocks + jnp.where(w < extra, 1, 0)
        sem_i = (sia, sib)
        sem_w = (swa, swb)

        def issue_idx(i, p):
            off = (w + i * _NW) * _CH
            pltpu.async_copy(idx.at[pl.ds(off, _CH)], idx_v.at[p], sem_i[p])

        def wait_idx(p):
            pltpu.make_async_copy(
                idx.at[pl.ds(0, _CH)], idx_v.at[p], sem_i[p]).wait()

        def do_gathers(p):
            cps = [
                pltpu.async_copy(
                    table.at[idx_v.at[p].at[pl.ds(j * _ST, _ST)]],
                    rows_v.at[p].at[pl.ds(j * _ST, _ST)],
                    sg,
                )
                for j in range(_K)
            ]
            for cp in cps:
                cp.wait()

        def issue_wb(i, p):
            off = (w + i * _NW) * _CH
            pltpu.async_copy(rows_v.at[p], out.at[pl.ds(off, _CH)], sem_w[p])

        def wait_wb(p):
            pltpu.make_async_copy(
                rows_v.at[p], out.at[pl.ds(0, _CH)], sem_w[p]).wait()

        @pl.when(nb > 0)
        def _():
            issue_idx(0, 0)

        def body(t, carry):
            b0 = 2 * t
            b1 = b0 + 1

            @pl.when(t > 0)
            def _():
                wait_wb(0)

            wait_idx(0)

            @pl.when(b1 < nb)
            def _():
                issue_idx(b1, 1)

            do_gathers(0)
            issue_wb(b0, 0)

            @pl.when(b1 < nb)
            def _():
                @pl.when(t > 0)
                def _():
                    wait_wb(1)

                wait_idx(1)

                @pl.when(b0 + 2 < nb)
                def _():
                    issue_idx(b0 + 2, 0)

                do_gathers(1)
                issue_wb(b1, 1)

            return carry

        lax.fori_loop(0, (nb + 1) // 2, body, 0)

        @pl.when(nb > 0)
        def _():
            wait_wb(0)

        @pl.when(nb > 1)
        def _():
            wait_wb(1)

    return gather_k


_CHS = 512
_KS = _CHS // _ST


@functools.cache
def _scatter_fn(n_edges, dim, n_rows):
    nblk = n_edges // _CHS
    base_blocks = nblk // _NW
    extra = nblk % _NW
    rows_per_tile = n_rows // _NS
    zch = 320
    while rows_per_tile % zch:
        zch //= 2
    zch = min(rows_per_tile, zch)
    n_zch = rows_per_tile // zch

    @functools.partial(
        pl.kernel,
        out_type=jax.ShapeDtypeStruct((_NC, n_rows, dim), jnp.float32),
        mesh=_mesh(),
        compiler_params=pltpu.CompilerParams(use_tc_tiling_on_sc=False),
        scratch_types=[
            pltpu.VMEM((2, _KS, _ST), jnp.int32),
            pltpu.VMEM((2, _CHS, dim), jnp.float32),
            pltpu.VMEM((zch, dim), jnp.float32),
            pltpu.VMEM_SHARED((n_rows, dim), jnp.float32),
            pltpu.SemaphoreType.DMA,
            pltpu.SemaphoreType.DMA,
        ],
    )
    def scatter_k(rows, idx2, out, idx_v, rows_v, zbuf, table, sa, sb):
        c = lax.axis_index("c")
        s_ax = lax.axis_index("s")
        w = _wid()
        sems = (sa, sb)

        def zero_body(i, carry):
            zbuf[i, :] = jnp.zeros((16,), jnp.float32)
            return carry

        lax.fori_loop(0, zch, zero_body, 0)
        for q in range(n_zch):
            pltpu.sync_copy(
                zbuf, table.at[pl.ds(s_ax * rows_per_tile + q * zch, zch)]
            )
        plsc.subcore_barrier()

        nb = base_blocks + jnp.where(w < extra, 1, 0)

        def issue_blk(i, p):
            blk = w + i * _NW
            pltpu.async_copy(
                idx2.at[pl.ds(blk * _KS, _KS)], idx_v.at[p], sems[p])
            pltpu.async_copy(
                rows.at[pl.ds(blk * _CHS, _CHS)], rows_v.at[p], sems[p])

        def wait_blk(p):
            pltpu.make_async_copy(
                idx2.at[pl.ds(0, _KS)], idx_v.at[p], sems[p]).wait()
            pltpu.make_async_copy(
                rows.at[pl.ds(0, _CHS)], rows_v.at[p], sems[p]).wait()

        def do_adds(p):
            for j in range(_KS):
                pltpu.sync_copy(
                    rows_v.at[p].at[pl.ds(j * _ST, _ST)],
                    table.at[idx_v.at[p].at[j]],
                    add=True,
                )

        @pl.when(nb > 0)
        def _():
            issue_blk(0, 0)

        def body(t, carry):
            b0 = 2 * t
            b1 = b0 + 1
            wait_blk(0)

            @pl.when(b1 < nb)
            def _():
                issue_blk(b1, 1)

            do_adds(0)

            @pl.when(b1 < nb)
            def _():
                wait_blk(1)

                @pl.when(b0 + 2 < nb)
                def _():
                    issue_blk(b0 + 2, 0)

                do_adds(1)

            return carry

        lax.fori_loop(0, (nb + 1) // 2, body, 0)
        plsc.subcore_barrier()
        for q in range(n_zch):
            r0 = s_ax * rows_per_tile + q * zch
            pltpu.sync_copy(table.at[pl.ds(r0, zch)], zbuf)
            pltpu.sync_copy(zbuf, out.at[c].at[pl.ds(r0, zch)])

    return scatter_k


def _dotd(a, b):
    return jnp.dot(
        a.astype(jnp.bfloat16), b.astype(jnp.bfloat16),
        preferred_element_type=jnp.float32,
    )


def _proj_body(x_ref, w_ref, b_ref, out_ref):
    out_ref[...] = _dotd(x_ref[...], w_ref[...]) + b_ref[...]


def _proj(x2, wbd, bt):
    n8, dk = x2.shape
    bn = 3200
    return pl.pallas_call(
        _proj_body,
        grid=(n8 // bn,),
        in_specs=[
            pl.BlockSpec((bn, dk), lambda i: (i, 0)),
            pl.BlockSpec((dk, 128), lambda i: (0, 0)),
            pl.BlockSpec((1, 128), lambda i: (0, 0)),
        ],
        out_specs=pl.BlockSpec((bn, 128), lambda i: (i, 0)),
        out_shape=jax.ShapeDtypeStruct((n8, 128), jnp.float32),
    )(x2, wbd, bt)


def _msg_body(g_ref, ef_ref, w1e_ref, w2_ref, b2_ref, out_ref):
    h1 = jnp.maximum(g_ref[...] + _dotd(ef_ref[...], w1e_ref[...]), 0.0)
    out_ref[...] = jnp.maximum(_dotd(h1, w2_ref[...]) + b2_ref[...], 0.0)


def _msg(g2, ef2, w1ebd, w2bd, b2t):
    e8, _ = g2.shape
    de = ef2.shape[1]
    be = 2000
    return pl.pallas_call(
        _msg_body,
        grid=(e8 // be,),
        in_specs=[
            pl.BlockSpec((be, 128), lambda i: (i, 0)),
            pl.BlockSpec((be, de), lambda i: (i, 0)),
            pl.BlockSpec((de, 128), lambda i: (0, 0)),
            pl.BlockSpec((128, 128), lambda i: (0, 0)),
            pl.BlockSpec((1, 128), lambda i: (0, 0)),
        ],
        out_specs=pl.BlockSpec((be, 128), lambda i: (i, 0)),
        out_shape=jax.ShapeDtypeStruct((e8, 128), jnp.float32),
    )(g2, ef2, w1ebd, w2bd, b2t)


def _upd_body(x_ref, a_ref, ux_ref, ua_ref, b_ref, out_ref):
    a = a_ref[0] + a_ref[1]
    out_ref[...] = jnp.maximum(
        _dotd(x_ref[...], ux_ref[...]) + _dotd(a, ua_ref[...]) + b_ref[...],
        0.0,
    )


def _update(x2, aggp2, uxbd, uabd, bt):
    n8, dk = x2.shape
    bn = 3200
    return pl.pallas_call(
        _upd_body,
        grid=(n8 // bn,),
        in_specs=[
            pl.BlockSpec((bn, dk), lambda i: (i, 0)),
            pl.BlockSpec((2, bn, 128), lambda i: (0, i, 0)),
            pl.BlockSpec((dk, 128), lambda i: (0, 0)),
            pl.BlockSpec((128, 128), lambda i: (0, 0)),
            pl.BlockSpec((1, 128), lambda i: (0, 0)),
        ],
        out_specs=pl.BlockSpec((bn, 128), lambda i: (i, 0)),
        out_shape=jax.ShapeDtypeStruct((n8, 128), jnp.float32),
    )(x2, aggp2, uxbd, uabd, bt)


def _pool_readout_body(
    y_ref, b_ref, wh_ref, bh_ref, wo_ref, bo_ref, out_ref, acc_ref
):
    i = pl.program_id(0)
    ng = acc_ref.shape[0]

    @pl.when(i == 0)
    def _():
        acc_ref[...] = jnp.zeros_like(acc_ref)

    bids = b_ref[0, 0, :]
    onehot_t = jnp.where(
        lax.broadcasted_iota(jnp.int32, (ng, bids.shape[0]), 0)
        == bids[None, :],
        1.0,
        0.0,
    )
    acc_ref[...] += jnp.dot(
        onehot_t, y_ref[...],
        preferred_element_type=jnp.float32, precision=lax.Precision.HIGHEST,
    )

    @pl.when(i == pl.num_programs(0) - 1)
    def _():
        h = jnp.maximum(_dotd(acc_ref[...], wh_ref[...]) + bh_ref[...], 0.0)
        out_ref[...] = _dotd(h, wo_ref[...]) + bo_ref[...]


def _pool_readout(y, batch3, wh, bh, wo, bo, n_graphs):
    n, dm = y.shape
    bn = 4096
    dh = wh.shape[1]
    return pl.pallas_call(
        _pool_readout_body,
        grid=(n // bn,),
        in_specs=[
            pl.BlockSpec((bn, dm), lambda i: (i, 0)),
            pl.BlockSpec((1, 1, bn), lambda i: (i, 0, 0)),
            pl.BlockSpec((dm, dh), lambda i: (0, 0)),
            pl.BlockSpec((1, dh), lambda i: (0, 0)),
            pl.BlockSpec((dh, 1), lambda i: (0, 0)),
            pl.BlockSpec((1, 1), lambda i: (0, 0)),
        ],
        out_specs=pl.BlockSpec((n_graphs, 1), lambda i: (0, 0)),
        out_shape=jax.ShapeDtypeStruct((n_graphs, 1), jnp.float32),
        scratch_shapes=[pltpu.VMEM((n_graphs, dm), jnp.float32)],
    )(y, batch3, wh, bh, wo, bo)


def kernel(node_features, edge_features, edge_idx, batch_idx, params):
    n_nodes, _ = node_features.shape
    n_edges = edge_features.shape[0]
    n_graphs = 128
    n_pad = 102400
    src = edge_idx[0]
    dst2 = edge_idx[1].reshape(n_edges // _ST, _ST)
    ef2 = edge_features.reshape(n_edges // 8, 24)

    def bd(w):
        k = w.shape[0]
        out = jnp.zeros((8, k, 8, 16), jnp.float32)
        out = out.at[jnp.arange(8), :, jnp.arange(8), :].set(
            jnp.broadcast_to(w, (8, k, 16)))
        return out.reshape(8 * k, 128)

    def tile_b(b):
        return jnp.tile(b.reshape(1, 16), (1, 8))

    x_pad = jnp.concatenate(
        [node_features,
         jnp.zeros((n_pad - n_nodes, node_features.shape[1]), jnp.float32)])
    y2 = x_pad.reshape(n_pad // 8, -1)
    for p in params["gnn"]:
        (w1, b1), (w2, b2) = p["M"]
        uw, ub = p["U"][0]
        din = y2.shape[1] // 8
        w1x, w1e = w1[:din], w1[din:]
        proj2 = _proj(y2, bd(w1x), tile_b(b1))
        g = _gather_fn(n_edges, 16, n_pad)(proj2.reshape(n_pad, 16), src)
        h2 = _msg(g.reshape(n_edges // 8, 128), ef2,
                  bd(w1e), bd(w2), tile_b(b2))
        aggp = _scatter_fn(n_edges, 16, n_pad)(h2.reshape(n_edges, 16), dst2)
        y2 = _update(y2, aggp.reshape(2, n_pad // 8, 128),
                     bd(uw[:din]), bd(uw[din:]), tile_b(ub))

    bpad = jnp.concatenate(
        [batch_idx.astype(jnp.int32),
         jnp.full((n_pad - n_nodes,), n_graphs, jnp.int32)])
    batch3 = bpad.reshape(n_pad // 4096, 1, 4096)
    return _pool_readout(
        y2.reshape(n_pad, 16),
        batch3,
        params["mlp_h_w"],
        params["mlp_h_b"].reshape(1, -1),
        params["mlp_o_w"],
        params["mlp_o_b"].reshape(1, -1),
        n_graphs,
    )


# speedup vs baseline: 5.2780x; 1.0144x over previous
import functools

import jax
import jax.numpy as jnp
from jax import lax
from jax.experimental import pallas as pl
from jax.experimental.pallas import tpu as pltpu
from jax.experimental.pallas import tpu_sc as plsc

_NC = 2
_NS = 16
_NW = _NC * _NS
_ST = 128
_CHS = 512
_KS = _CHS // _ST

@functools.cache
def _mesh():
    return plsc.VectorSubcoreMesh(
        core_axis_name="c", subcore_axis_name="s",
        num_cores=_NC, num_subcores=_NS,
    )


def _wid():
    return lax.axis_index("s") * _NC + lax.axis_index("c")


@functools.cache
def _gather_fn(n_edges, dim, n_rows):
    nblk = n_edges // _CHS
    base_blocks = nblk // _NW
    extra = nblk % _NW
    rows_per_tile = n_rows // _NS
    zch = 320
    while rows_per_tile % zch:
        zch //= 2
    n_zch = rows_per_tile // zch

    @functools.partial(
        pl.kernel,
        out_type=jax.ShapeDtypeStruct((n_edges, dim), jnp.float32),
        mesh=_mesh(),
        compiler_params=pltpu.CompilerParams(use_tc_tiling_on_sc=False),
        scratch_types=[
            pltpu.VMEM((2, _CHS), jnp.int32),
            pltpu.VMEM((2, _CHS, dim), jnp.float32),
            pltpu.VMEM((zch, dim), jnp.float32),
            pltpu.VMEM_SHARED((n_rows, dim), jnp.float32),
            pltpu.SemaphoreType.DMA,
            pltpu.SemaphoreType.DMA,
            pltpu.SemaphoreType.DMA,
            pltpu.SemaphoreType.DMA,
            pltpu.SemaphoreType.DMA,
        ],
    )
    def gather_k(table, idx, out, idx_v, rows_v, zbuf, table_s,
                 sia, sib, sg, swa, swb):
        w = _wid()
        s_ax = lax.axis_index("s")
        nb = base_blocks + jnp.where(w < extra, 1, 0)
        sem_i = (sia, sib)
        sem_w = (swa, swb)

        for q in range(n_zch):
            r0 = s_ax * rows_per_tile + q * zch
            pltpu.sync_copy(table.at[pl.ds(r0, zch)], zbuf)
            pltpu.sync_copy(zbuf, table_s.at[pl.ds(r0, zch)])
        plsc.subcore_barrier()

        def issue_idx(i, p):
            off = (w + i * _NW) * _CHS
            pltpu.async_copy(idx.at[pl.ds(off, _CHS)], idx_v.at[p], sem_i[p])

        def wait_idx(p):
            pltpu.make_async_copy(
                idx.at[pl.ds(0, _CHS)], idx_v.at[p], sem_i[p]).wait()

        def do_gathers(p):
            cps = [
                pltpu.async_copy(
                    table_s.at[idx_v.at[p].at[pl.ds(j * _ST, _ST)]],
                    rows_v.at[p].at[pl.ds(j * _ST, _ST)],
                    sg,
                )
                for j in range(_KS)
            ]
            for cp in cps:
                cp.wait()

        def issue_wb(i, p):
            off = (w + i * _NW) * _CHS
            pltpu.async_copy(rows_v.at[p], out.at[pl.ds(off, _CHS)], sem_w[p])

        def wait_wb(p):
            pltpu.make_async_copy(
                rows_v.at[p], out.at[pl.ds(0, _CHS)], sem_w[p]).wait()

        @pl.when(nb > 0)
        def _():
            issue_idx(0, 0)

        def body(t, carry):
            b0 = 2 * t
            b1 = b0 + 1

            @pl.when(t > 0)
            def _():
                wait_wb(0)

            wait_idx(0)

            @pl.when(b1 < nb)
            def _():
                issue_idx(b1, 1)

            do_gathers(0)
            issue_wb(b0, 0)

            @pl.when(b1 < nb)
            def _():
                @pl.when(t > 0)
                def _():
                    wait_wb(1)

                wait_idx(1)

                @pl.when(b0 + 2 < nb)
                def _():
                    issue_idx(b0 + 2, 0)

                do_gathers(1)
                issue_wb(b1, 1)

            return carry

        lax.fori_loop(0, (nb + 1) // 2, body, 0)

        @pl.when(nb > 0)
        def _():
            wait_wb(0)

        @pl.when(nb > 1)
        def _():
            wait_wb(1)

    return gather_k


@functools.cache
def _scatter_fn(n_edges, dim, n_rows):
    nblk = n_edges // _CHS
    base_blocks = nblk // _NW
    extra = nblk % _NW
    rows_per_tile = n_rows // _NS
    zch = 320
    while rows_per_tile % zch:
        zch //= 2
    zch = min(rows_per_tile, zch)
    n_zch = rows_per_tile // zch

    @functools.partial(
        pl.kernel,
        out_type=jax.ShapeDtypeStruct((_NC, n_rows, dim), jnp.float32),
        mesh=_mesh(),
        compiler_params=pltpu.CompilerParams(use_tc_tiling_on_sc=False),
        scratch_types=[
            pltpu.VMEM((2, _KS, _ST), jnp.int32),
            pltpu.VMEM((2, _CHS, dim), jnp.float32),
            pltpu.VMEM((zch, dim), jnp.float32),
            pltpu.VMEM_SHARED((n_rows, dim), jnp.float32),
            pltpu.SemaphoreType.DMA,
            pltpu.SemaphoreType.DMA,
            pltpu.SemaphoreType.DMA,
        ],
    )
    def scatter_k(rows, idx2, out, idx_v, rows_v, zbuf, table, sa, sb, sg):
        c = lax.axis_index("c")
        s_ax = lax.axis_index("s")
        w = _wid()
        sems = (sa, sb)

        def zero_body(i, carry):
            zbuf[i, :] = jnp.zeros((16,), jnp.float32)
            return carry

        lax.fori_loop(0, zch, zero_body, 0)
        for q in range(n_zch):
            pltpu.sync_copy(
                zbuf, table.at[pl.ds(s_ax * rows_per_tile + q * zch, zch)]
            )
        plsc.subcore_barrier()

        nb = base_blocks + jnp.where(w < extra, 1, 0)

        def issue_blk(i, p):
            blk = w + i * _NW
            pltpu.async_copy(
                idx2.at[pl.ds(blk * _KS, _KS)], idx_v.at[p], sems[p])
            pltpu.async_copy(
                rows.at[pl.ds(blk * _CHS, _CHS)], rows_v.at[p], sems[p])

        def wait_blk(p):
            pltpu.make_async_copy(
                idx2.at[pl.ds(0, _KS)], idx_v.at[p], sems[p]).wait()
            pltpu.make_async_copy(
                rows.at[pl.ds(0, _CHS)], rows_v.at[p], sems[p]).wait()

        def do_adds(p):
            cps = [
                pltpu.async_copy(
                    rows_v.at[p].at[pl.ds(j * _ST, _ST)],
                    table.at[idx_v.at[p].at[j]],
                    sg,
                    add=True,
                )
                for j in range(_KS)
            ]
            for cp in cps:
                cp.wait()

        @pl.when(nb > 0)
        def _():
            issue_blk(0, 0)

        def body(t, carry):
            b0 = 2 * t
            b1 = b0 + 1
            wait_blk(0)

            @pl.when(b1 < nb)
            def _():
                issue_blk(b1, 1)

            do_adds(0)

            @pl.when(b1 < nb)
            def _():
                wait_blk(1)

                @pl.when(b0 + 2 < nb)
                def _():
                    issue_blk(b0 + 2, 0)

                do_adds(1)

            return carry

        lax.fori_loop(0, (nb + 1) // 2, body, 0)
        plsc.subcore_barrier()
        for q in range(n_zch):
            r0 = s_ax * rows_per_tile + q * zch
            pltpu.sync_copy(table.at[pl.ds(r0, zch)], zbuf)
            pltpu.sync_copy(zbuf, out.at[c].at[pl.ds(r0, zch)])

    return scatter_k


def _dotd(a, b):
    return jnp.dot(
        a.astype(jnp.bfloat16), b.astype(jnp.bfloat16),
        preferred_element_type=jnp.float32,
    )


def _proj_body(x_ref, w_ref, b_ref, out_ref):
    out_ref[...] = _dotd(x_ref[...], w_ref[...]) + b_ref[...]


def _proj(x2, wbd, bt):
    n8, dk = x2.shape
    bn = 3200
    return pl.pallas_call(
        _proj_body,
        grid=(n8 // bn,),
        in_specs=[
            pl.BlockSpec((bn, dk), lambda i: (i, 0)),
            pl.BlockSpec((dk, 128), lambda i: (0, 0)),
            pl.BlockSpec((1, 128), lambda i: (0, 0)),
        ],
        out_specs=pl.BlockSpec((bn, 128), lambda i: (i, 0)),
        out_shape=jax.ShapeDtypeStruct((n8, 128), jnp.float32),
    )(x2, wbd, bt)


def _msg_body(g_ref, ef_ref, w1e_ref, w2_ref, b2_ref, out_ref):
    h1 = jnp.maximum(g_ref[...] + _dotd(ef_ref[...], w1e_ref[...]), 0.0)
    out_ref[...] = jnp.maximum(_dotd(h1, w2_ref[...]) + b2_ref[...], 0.0)


def _msg(g2, ef2, w1ebd, w2bd, b2t):
    e8, _ = g2.shape
    de = ef2.shape[1]
    be = 2000
    return pl.pallas_call(
        _msg_body,
        grid=(e8 // be,),
        in_specs=[
            pl.BlockSpec((be, 128), lambda i: (i, 0)),
            pl.BlockSpec((be, de), lambda i: (i, 0)),
            pl.BlockSpec((de, 128), lambda i: (0, 0)),
            pl.BlockSpec((128, 128), lambda i: (0, 0)),
            pl.BlockSpec((1, 128), lambda i: (0, 0)),
        ],
        out_specs=pl.BlockSpec((be, 128), lambda i: (i, 0)),
        out_shape=jax.ShapeDtypeStruct((e8, 128), jnp.float32),
    )(g2, ef2, w1ebd, w2bd, b2t)


def _upd_body(x_ref, a_ref, ux_ref, ua_ref, b_ref, out_ref):
    a = a_ref[0] + a_ref[1]
    out_ref[...] = jnp.maximum(
        _dotd(x_ref[...], ux_ref[...]) + _dotd(a, ua_ref[...]) + b_ref[...],
        0.0,
    )


def _update(x2, aggp2, uxbd, uabd, bt):
    n8, dk = x2.shape
    bn = 3200
    return pl.pallas_call(
        _upd_body,
        grid=(n8 // bn,),
        in_specs=[
            pl.BlockSpec((bn, dk), lambda i: (i, 0)),
            pl.BlockSpec((2, bn, 128), lambda i: (0, i, 0)),
            pl.BlockSpec((dk, 128), lambda i: (0, 0)),
            pl.BlockSpec((128, 128), lambda i: (0, 0)),
            pl.BlockSpec((1, 128), lambda i: (0, 0)),
        ],
        out_specs=pl.BlockSpec((bn, 128), lambda i: (i, 0)),
        out_shape=jax.ShapeDtypeStruct((n8, 128), jnp.float32),
    )(x2, aggp2, uxbd, uabd, bt)


def _pool_readout_body(
    y_ref, b_ref, wh_ref, bh_ref, wo_ref, bo_ref, out_ref, acc_ref
):
    i = pl.program_id(0)
    ng = acc_ref.shape[0]

    @pl.when(i == 0)
    def _():
        acc_ref[...] = jnp.zeros_like(acc_ref)

    bids = b_ref[0, 0, :]
    onehot_t = jnp.where(
        lax.broadcasted_iota(jnp.int32, (ng, bids.shape[0]), 0)
        == bids[None, :],
        1.0,
        0.0,
    )
    acc_ref[...] += jnp.dot(
        onehot_t, y_ref[...],
        preferred_element_type=jnp.float32, precision=lax.Precision.HIGHEST,
    )

    @pl.when(i == pl.num_programs(0) - 1)
    def _():
        h = jnp.maximum(_dotd(acc_ref[...], wh_ref[...]) + bh_ref[...], 0.0)
        out_ref[...] = _dotd(h, wo_ref[...]) + bo_ref[...]


def _pool_readout(y, batch3, wh, bh, wo, bo, n_graphs):
    n, dm = y.shape
    bn = 4096
    dh = wh.shape[1]
    return pl.pallas_call(
        _pool_readout_body,
        grid=(n // bn,),
        in_specs=[
            pl.BlockSpec((bn, dm), lambda i: (i, 0)),
            pl.BlockSpec((1, 1, bn), lambda i: (i, 0, 0)),
            pl.BlockSpec((dm, dh), lambda i: (0, 0)),
            pl.BlockSpec((1, dh), lambda i: (0, 0)),
            pl.BlockSpec((dh, 1), lambda i: (0, 0)),
            pl.BlockSpec((1, 1), lambda i: (0, 0)),
        ],
        out_specs=pl.BlockSpec((n_graphs, 1), lambda i: (0, 0)),
        out_shape=jax.ShapeDtypeStruct((n_graphs, 1), jnp.float32),
        scratch_shapes=[pltpu.VMEM((n_graphs, dm), jnp.float32)],
    )(y, batch3, wh, bh, wo, bo)


def kernel(node_features, edge_features, edge_idx, batch_idx, params):
    n_nodes, _ = node_features.shape
    n_edges = edge_features.shape[0]
    n_graphs = 128
    n_pad = 102400
    src = edge_idx[0]
    dst2 = edge_idx[1].reshape(n_edges // _ST, _ST)
    ef2 = edge_features.reshape(n_edges // 8, 24)

    def bd(w):
        k = w.shape[0]
        out = jnp.zeros((8, k, 8, 16), jnp.float32)
        out = out.at[jnp.arange(8), :, jnp.arange(8), :].set(
            jnp.broadcast_to(w, (8, k, 16)))
        return out.reshape(8 * k, 128)

    def tile_b(b):
        return jnp.tile(b.reshape(1, 16), (1, 8))

    x_pad = jnp.concatenate(
        [node_features,
         jnp.zeros((n_pad - n_nodes, node_features.shape[1]), jnp.float32)])
    y2 = x_pad.reshape(n_pad // 8, -1)
    for p in params["gnn"]:
        (w1, b1), (w2, b2) = p["M"]
        uw, ub = p["U"][0]
        din = y2.shape[1] // 8
        w1x, w1e = w1[:din], w1[din:]
        proj2 = _proj(y2, bd(w1x), tile_b(b1))
        g = _gather_fn(n_edges, 16, n_pad)(proj2.reshape(n_pad, 16), src)
        h2 = _msg(g.reshape(n_edges // 8, 128), ef2,
                  bd(w1e), bd(w2), tile_b(b2))
        aggp = _scatter_fn(n_edges, 16, n_pad)(h2.reshape(n_edges, 16), dst2)
        y2 = _update(y2, aggp.reshape(2, n_pad // 8, 128),
                     bd(uw[:din]), bd(uw[din:]), tile_b(ub))

    bpad = jnp.concatenate(
        [batch_idx.astype(jnp.int32),
         jnp.full((n_pad - n_nodes,), n_graphs, jnp.int32)])
    batch3 = bpad.reshape(n_pad // 4096, 1, 4096)
    return _pool_readout(
        y2.reshape(n_pad, 16),
        batch3,
        params["mlp_h_w"],
        params["mlp_h_b"].reshape(1, -1),
        params["mlp_o_w"],
        params["mlp_o_b"].reshape(1, -1),
        n_graphs,
    )


# speedup vs baseline: 5.2781x; 1.0000x over previous
import functools

import jax
import jax.numpy as jnp
from jax import lax
from jax.experimental import pallas as pl
from jax.experimental.pallas import tpu as pltpu
from jax.experimental.pallas import tpu_sc as plsc

_NC = 2
_NS = 16
_NW = _NC * _NS
_ST = 128
_CHS = 512
_KS = _CHS // _ST

@functools.cache
def _mesh():
    return plsc.VectorSubcoreMesh(
        core_axis_name="c", subcore_axis_name="s",
        num_cores=_NC, num_subcores=_NS,
    )


def _wid():
    return lax.axis_index("s") * _NC + lax.axis_index("c")


@functools.cache
def _gather_fn(n_edges, dim, n_rows):
    nblk = n_edges // _CHS
    base_blocks = nblk // _NW
    extra = nblk % _NW
    rows_per_tile = n_rows // _NS
    zch = 320
    while rows_per_tile % zch:
        zch //= 2
    n_zch = rows_per_tile // zch

    @functools.partial(
        pl.kernel,
        out_type=jax.ShapeDtypeStruct((n_edges, dim), jnp.float32),
        mesh=_mesh(),
        compiler_params=pltpu.CompilerParams(use_tc_tiling_on_sc=False),
        scratch_types=[
            pltpu.VMEM((2, _CHS), jnp.int32),
            pltpu.VMEM((2, _CHS, dim), jnp.float32),
            pltpu.VMEM((zch, dim), jnp.float32),
            pltpu.VMEM_SHARED((n_rows, dim), jnp.float32),
            pltpu.SemaphoreType.DMA,
            pltpu.SemaphoreType.DMA,
            pltpu.SemaphoreType.DMA,
            pltpu.SemaphoreType.DMA,
            pltpu.SemaphoreType.DMA,
        ],
    )
    def gather_k(table, idx, out, idx_v, rows_v, zbuf, table_s,
                 sia, sib, sg, swa, swb):
        w = _wid()
        s_ax = lax.axis_index("s")
        nb = base_blocks + jnp.where(w < extra, 1, 0)
        sem_i = (sia, sib)
        sem_w = (swa, swb)

        for q in range(n_zch):
            r0 = s_ax * rows_per_tile + q * zch
            pltpu.sync_copy(table.at[pl.ds(r0, zch)], zbuf)
            pltpu.sync_copy(zbuf, table_s.at[pl.ds(r0, zch)])
        plsc.subcore_barrier()

        def issue_idx(i, p):
            off = (w + i * _NW) * _CHS
            pltpu.async_copy(idx.at[pl.ds(off, _CHS)], idx_v.at[p], sem_i[p])

        def wait_idx(p):
            pltpu.make_async_copy(
                idx.at[pl.ds(0, _CHS)], idx_v.at[p], sem_i[p]).wait()

        def do_gathers(p):
            pltpu.async_copy(
                table_s.at[idx_v.at[p]], rows_v.at[p], sg).wait()

        def issue_wb(i, p):
            off = (w + i * _NW) * _CHS
            pltpu.async_copy(rows_v.at[p], out.at[pl.ds(off, _CHS)], sem_w[p])

        def wait_wb(p):
            pltpu.make_async_copy(
                rows_v.at[p], out.at[pl.ds(0, _CHS)], sem_w[p]).wait()

        @pl.when(nb > 0)
        def _():
            issue_idx(0, 0)

        def body(t, carry):
            b0 = 2 * t
            b1 = b0 + 1

            @pl.when(t > 0)
            def _():
                wait_wb(0)

            wait_idx(0)

            @pl.when(b1 < nb)
            def _():
                issue_idx(b1, 1)

            do_gathers(0)
            issue_wb(b0, 0)

            @pl.when(b1 < nb)
            def _():
                @pl.when(t > 0)
                def _():
                    wait_wb(1)

                wait_idx(1)

                @pl.when(b0 + 2 < nb)
                def _():
                    issue_idx(b0 + 2, 0)

                do_gathers(1)
                issue_wb(b1, 1)

            return carry

        lax.fori_loop(0, (nb + 1) // 2, body, 0)

        @pl.when(nb > 0)
        def _():
            wait_wb(0)

        @pl.when(nb > 1)
        def _():
            wait_wb(1)

    return gather_k


@functools.cache
def _scatter_fn(n_edges, dim, n_rows):
    nblk = n_edges // _CHS
    base_blocks = nblk // _NW
    extra = nblk % _NW
    rows_per_tile = n_rows // _NS
    zch = 320
    while rows_per_tile % zch:
        zch //= 2
    zch = min(rows_per_tile, zch)
    n_zch = rows_per_tile // zch

    @functools.partial(
        pl.kernel,
        out_type=jax.ShapeDtypeStruct((_NC, n_rows, dim), jnp.float32),
        mesh=_mesh(),
        compiler_params=pltpu.CompilerParams(use_tc_tiling_on_sc=False),
        scratch_types=[
            pltpu.VMEM((2, _CHS), jnp.int32),
            pltpu.VMEM((2, _CHS, dim), jnp.float32),
            pltpu.VMEM((zch, dim), jnp.float32),
            pltpu.VMEM_SHARED((n_rows, dim), jnp.float32),
            pltpu.SemaphoreType.DMA,
            pltpu.SemaphoreType.DMA,
            pltpu.SemaphoreType.DMA,
        ],
    )
    def scatter_k(rows, idx2, out, idx_v, rows_v, zbuf, table, sa, sb, sg):
        c = lax.axis_index("c")
        s_ax = lax.axis_index("s")
        w = _wid()
        sems = (sa, sb)

        def zero_body(i, carry):
            zbuf[i, :] = jnp.zeros((16,), jnp.float32)
            return carry

        lax.fori_loop(0, zch, zero_body, 0)
        for q in range(n_zch):
            pltpu.sync_copy(
                zbuf, table.at[pl.ds(s_ax * rows_per_tile + q * zch, zch)]
            )
        plsc.subcore_barrier()

        nb = base_blocks + jnp.where(w < extra, 1, 0)

        def issue_blk(i, p):
            blk = w + i * _NW
            pltpu.async_copy(
                idx2.at[pl.ds(blk * _CHS, _CHS)], idx_v.at[p], sems[p])
            pltpu.async_copy(
                rows.at[pl.ds(blk * _CHS, _CHS)], rows_v.at[p], sems[p])

        def wait_blk(p):
            pltpu.make_async_copy(
                idx2.at[pl.ds(0, _CHS)], idx_v.at[p], sems[p]).wait()
            pltpu.make_async_copy(
                rows.at[pl.ds(0, _CHS)], rows_v.at[p], sems[p]).wait()

        def do_adds(p):
            pltpu.async_copy(
                rows_v.at[p], table.at[idx_v.at[p]], sg, add=True).wait()

        @pl.when(nb > 0)
        def _():
            issue_blk(0, 0)

        def body(t, carry):
            b0 = 2 * t
            b1 = b0 + 1
            wait_blk(0)

            @pl.when(b1 < nb)
            def _():
                issue_blk(b1, 1)

            do_adds(0)

            @pl.when(b1 < nb)
            def _():
                wait_blk(1)

                @pl.when(b0 + 2 < nb)
                def _():
                    issue_blk(b0 + 2, 0)

                do_adds(1)

            return carry

        lax.fori_loop(0, (nb + 1) // 2, body, 0)
        plsc.subcore_barrier()
        for q in range(n_zch):
            r0 = s_ax * rows_per_tile + q * zch
            pltpu.sync_copy(table.at[pl.ds(r0, zch)], zbuf)
            pltpu.sync_copy(zbuf, out.at[c].at[pl.ds(r0, zch)])

    return scatter_k


def _dotd(a, b):
    return jnp.dot(
        a.astype(jnp.bfloat16), b.astype(jnp.bfloat16),
        preferred_element_type=jnp.float32,
    )


def _proj_body(x_ref, w_ref, b_ref, out_ref):
    out_ref[...] = _dotd(x_ref[...], w_ref[...]) + b_ref[...]


def _proj(x2, wbd, bt):
    n8, dk = x2.shape
    bn = 3200
    return pl.pallas_call(
        _proj_body,
        grid=(n8 // bn,),
        in_specs=[
            pl.BlockSpec((bn, dk), lambda i: (i, 0)),
            pl.BlockSpec((dk, 128), lambda i: (0, 0)),
            pl.BlockSpec((1, 128), lambda i: (0, 0)),
        ],
        out_specs=pl.BlockSpec((bn, 128), lambda i: (i, 0)),
        out_shape=jax.ShapeDtypeStruct((n8, 128), jnp.float32),
    )(x2, wbd, bt)


def _msg_body(g_ref, ef_ref, w1e_ref, w2_ref, b2_ref, out_ref):
    h1 = jnp.maximum(g_ref[...] + _dotd(ef_ref[...], w1e_ref[...]), 0.0)
    out_ref[...] = jnp.maximum(_dotd(h1, w2_ref[...]) + b2_ref[...], 0.0)


def _msg(g2, ef2, w1ebd, w2bd, b2t):
    e8, _ = g2.shape
    de = ef2.shape[1]
    be = 2000
    return pl.pallas_call(
        _msg_body,
        grid=(e8 // be,),
        in_specs=[
            pl.BlockSpec((be, 128), lambda i: (i, 0)),
            pl.BlockSpec((be, de), lambda i: (i, 0)),
            pl.BlockSpec((de, 128), lambda i: (0, 0)),
            pl.BlockSpec((128, 128), lambda i: (0, 0)),
            pl.BlockSpec((1, 128), lambda i: (0, 0)),
        ],
        out_specs=pl.BlockSpec((be, 128), lambda i: (i, 0)),
        out_shape=jax.ShapeDtypeStruct((e8, 128), jnp.float32),
    )(g2, ef2, w1ebd, w2bd, b2t)


def _upd_body(x_ref, a_ref, ux_ref, ua_ref, b_ref, out_ref):
    a = a_ref[0] + a_ref[1]
    out_ref[...] = jnp.maximum(
        _dotd(x_ref[...], ux_ref[...]) + _dotd(a, ua_ref[...]) + b_ref[...],
        0.0,
    )


def _update(x2, aggp2, uxbd, uabd, bt):
    n8, dk = x2.shape
    bn = 3200
    return pl.pallas_call(
        _upd_body,
        grid=(n8 // bn,),
        in_specs=[
            pl.BlockSpec((bn, dk), lambda i: (i, 0)),
            pl.BlockSpec((2, bn, 128), lambda i: (0, i, 0)),
            pl.BlockSpec((dk, 128), lambda i: (0, 0)),
            pl.BlockSpec((128, 128), lambda i: (0, 0)),
            pl.BlockSpec((1, 128), lambda i: (0, 0)),
        ],
        out_specs=pl.BlockSpec((bn, 128), lambda i: (i, 0)),
        out_shape=jax.ShapeDtypeStruct((n8, 128), jnp.float32),
    )(x2, aggp2, uxbd, uabd, bt)


def _pool_readout_body(
    y_ref, b_ref, wh_ref, bh_ref, wo_ref, bo_ref, out_ref, acc_ref
):
    i = pl.program_id(0)
    ng = acc_ref.shape[0]

    @pl.when(i == 0)
    def _():
        acc_ref[...] = jnp.zeros_like(acc_ref)

    bids = b_ref[0, 0, :]
    onehot_t = jnp.where(
        lax.broadcasted_iota(jnp.int32, (ng, bids.shape[0]), 0)
        == bids[None, :],
        1.0,
        0.0,
    )
    acc_ref[...] += jnp.dot(
        onehot_t, y_ref[...],
        preferred_element_type=jnp.float32, precision=lax.Precision.HIGHEST,
    )

    @pl.when(i == pl.num_programs(0) - 1)
    def _():
        h = jnp.maximum(_dotd(acc_ref[...], wh_ref[...]) + bh_ref[...], 0.0)
        out_ref[...] = _dotd(h, wo_ref[...]) + bo_ref[...]


def _pool_readout(y, batch3, wh, bh, wo, bo, n_graphs):
    n, dm = y.shape
    bn = 4096
    dh = wh.shape[1]
    return pl.pallas_call(
        _pool_readout_body,
        grid=(n // bn,),
        in_specs=[
            pl.BlockSpec((bn, dm), lambda i: (i, 0)),
            pl.BlockSpec((1, 1, bn), lambda i: (i, 0, 0)),
            pl.BlockSpec((dm, dh), lambda i: (0, 0)),
            pl.BlockSpec((1, dh), lambda i: (0, 0)),
            pl.BlockSpec((dh, 1), lambda i: (0, 0)),
            pl.BlockSpec((1, 1), lambda i: (0, 0)),
        ],
        out_specs=pl.BlockSpec((n_graphs, 1), lambda i: (0, 0)),
        out_shape=jax.ShapeDtypeStruct((n_graphs, 1), jnp.float32),
        scratch_shapes=[pltpu.VMEM((n_graphs, dm), jnp.float32)],
    )(y, batch3, wh, bh, wo, bo)


def kernel(node_features, edge_features, edge_idx, batch_idx, params):
    n_nodes, _ = node_features.shape
    n_edges = edge_features.shape[0]
    n_graphs = 128
    n_pad = 102400
    src = edge_idx[0]
    dst = edge_idx[1]
    ef2 = edge_features.reshape(n_edges // 8, 24)

    def bd(w):
        k = w.shape[0]
        out = jnp.zeros((8, k, 8, 16), jnp.float32)
        out = out.at[jnp.arange(8), :, jnp.arange(8), :].set(
            jnp.broadcast_to(w, (8, k, 16)))
        return out.reshape(8 * k, 128)

    def tile_b(b):
        return jnp.tile(b.reshape(1, 16), (1, 8))

    x_pad = jnp.concatenate(
        [node_features,
         jnp.zeros((n_pad - n_nodes, node_features.shape[1]), jnp.float32)])
    y2 = x_pad.reshape(n_pad // 8, -1)
    for p in params["gnn"]:
        (w1, b1), (w2, b2) = p["M"]
        uw, ub = p["U"][0]
        din = y2.shape[1] // 8
        w1x, w1e = w1[:din], w1[din:]
        proj2 = _proj(y2, bd(w1x), tile_b(b1))
        g = _gather_fn(n_edges, 16, n_pad)(proj2.reshape(n_pad, 16), src)
        h2 = _msg(g.reshape(n_edges // 8, 128), ef2,
                  bd(w1e), bd(w2), tile_b(b2))
        aggp = _scatter_fn(n_edges, 16, n_pad)(h2.reshape(n_edges, 16), dst)
        y2 = _update(y2, aggp.reshape(2, n_pad // 8, 128),
                     bd(uw[:din]), bd(uw[din:]), tile_b(ub))

    bpad = jnp.concatenate(
        [batch_idx.astype(jnp.int32),
         jnp.full((n_pad - n_nodes,), n_graphs, jnp.int32)])
    batch3 = bpad.reshape(n_pad // 4096, 1, 4096)
    return _pool_readout(
        y2.reshape(n_pad, 16),
        batch3,
        params["mlp_h_w"],
        params["mlp_h_b"].reshape(1, -1),
        params["mlp_o_w"],
        params["mlp_o_b"].reshape(1, -1),
        n_graphs,
    )
